# Initial kernel scaffold; baseline (speedup 1.0000x reference)
import jax
import jax.numpy as jnp
from jax.experimental import pallas as pl


def kernel(x, edge_index, W_emb1, b_emb1, W_emb2, b_emb2, W_lin, b_lin, W_gat, att_src, att_dst, b_gat):
    raise NotImplementedError("write your pallas kernel here")


# speedup vs baseline: 22.9801x; 22.9801x over previous
import functools

import jax
import jax.numpy as jnp
from jax import lax
from jax.experimental import pallas as pl
from jax.experimental.pallas import tpu as pltpu
from jax.experimental.pallas import tpu_sc as plsc

N_NODES = 10000
N_EDGES = 160000
D_IN = 256
D_HID1 = 512
HID = 256
HEADS = 8
NCLS = 64
DGAT = HEADS * NCLS

ROWBLK = 256
NPAD = 10240
NBLK = NPAD // ROWBLK
EPAD = 163840
EROWS = EPAD // 128
SENT = N_NODES

NC, NS = 2, 16
NW = NC * NS
WROWS = EROWS // NW


def _dense_body(x_ref, w1_ref, b1_ref, w2_ref, b2_ref, wl_ref, bl_ref,
                wg_ref, asel_ref, dsel_ref, esel_ref,
                xw_ref, atabs_ref, atabd_ref, bacc_ref):
    i = pl.program_id(0)
    x = x_ref[...]
    h = jnp.maximum(
        jnp.dot(x, w1_ref[...], preferred_element_type=jnp.float32)
        + b1_ref[...], 0.0)
    h = jnp.dot(h, w2_ref[...], preferred_element_type=jnp.float32) + b2_ref[...]
    h = jnp.dot(h, wl_ref[...], preferred_element_type=jnp.float32) + bl_ref[...]
    xw = jnp.dot(h, wg_ref[...], preferred_element_type=jnp.float32)
    xw_ref[...] = xw
    asrc = jnp.dot(xw * asel_ref[...], esel_ref[...],
                   preferred_element_type=jnp.float32)
    adst = jnp.dot(xw * dsel_ref[...], esel_ref[...],
                   preferred_element_type=jnp.float32)
    row = i * ROWBLK + lax.broadcasted_iota(jnp.int32, (ROWBLK, 1), 0)
    valid = row < N_NODES
    neg = jnp.full_like(asrc, -1e30)
    asrc_m = jnp.where(valid, asrc, neg)
    adst_m = jnp.where(valid, adst, neg)
    zpad = jnp.zeros((ROWBLK, 8), jnp.float32)
    atabs_ref[...] = jnp.concatenate([asrc_m, zpad], axis=1)
    atabd_ref[...] = jnp.concatenate([adst_m, zpad], axis=1)
    cur = jnp.concatenate(
        [jnp.max(asrc_m, axis=0, keepdims=True),
         jnp.max(adst_m, axis=0, keepdims=True)], axis=0)
    cur16 = jnp.concatenate([cur, jnp.zeros((2, 8), jnp.float32)], axis=1)
    prev = jnp.where(i == 0, jnp.full((2, 16), -1e30), bacc_ref[...])
    bacc_ref[...] = jnp.maximum(prev, cur16)


_dense = pl.pallas_call(
    _dense_body,
    grid=(NBLK,),
    in_specs=[
        pl.BlockSpec((ROWBLK, D_IN), lambda i: (i, 0)),
        pl.BlockSpec((D_IN, D_HID1), lambda i: (0, 0)),
        pl.BlockSpec((1, D_HID1), lambda i: (0, 0)),
        pl.BlockSpec((D_HID1, HID), lambda i: (0, 0)),
        pl.BlockSpec((1, HID), lambda i: (0, 0)),
        pl.BlockSpec((HID, HID), lambda i: (0, 0)),
        pl.BlockSpec((1, HID), lambda i: (0, 0)),
        pl.BlockSpec((HID, DGAT), lambda i: (0, 0)),
        pl.BlockSpec((1, DGAT), lambda i: (0, 0)),
        pl.BlockSpec((1, DGAT), lambda i: (0, 0)),
        pl.BlockSpec((DGAT, HEADS), lambda i: (0, 0)),
    ],
    out_specs=[
        pl.BlockSpec((ROWBLK, DGAT), lambda i: (i, 0)),
        pl.BlockSpec((ROWBLK, 16), lambda i: (i, 0)),
        pl.BlockSpec((ROWBLK, 16), lambda i: (i, 0)),
        pl.BlockSpec((2, 16), lambda i: (0, 0)),
    ],
    out_shape=[
        jax.ShapeDtypeStruct((NPAD, DGAT), jnp.float32),
        jax.ShapeDtypeStruct((NPAD, 16), jnp.float32),
        jax.ShapeDtypeStruct((NPAD, 16), jnp.float32),
        jax.ShapeDtypeStruct((2, 16), jnp.float32),
    ],
)


_sc_mesh = plsc.VectorSubcoreMesh(core_axis_name="c", subcore_axis_name="s")


@functools.partial(
    pl.kernel,
    out_type=(
        jax.ShapeDtypeStruct((EPAD, 16), jnp.float32),
        jax.ShapeDtypeStruct((NPAD, 16), jnp.float32),
        jax.ShapeDtypeStruct((NPAD, 16), jnp.float32),
    ),
    mesh=_sc_mesh,
    compiler_params=pltpu.CompilerParams(use_tc_tiling_on_sc=False),
    scratch_types=[
        pltpu.VMEM((8, 128), jnp.int32),
        pltpu.VMEM((8, 128), jnp.int32),
        pltpu.VMEM((1024, 16), jnp.float32),
        pltpu.VMEM((1024, 16), jnp.float32),
        pltpu.VMEM((1024, 16), jnp.float32),
        pltpu.VMEM((2, 16), jnp.float32),
        pltpu.VMEM((640, 16), jnp.float32),
        pltpu.VMEM_SHARED((NPAD, 16), jnp.float32),
        pltpu.SemaphoreType.DMA,
    ],
)
def _sc_b1(srcm, dstm, atabs, atabd, bacc, w_hbm, d0_hbm, d1_hbm,
           sidx, didx, g1, g2, wbuf, bvm, zbuf, denom_sh, sem):
    cid = lax.axis_index("c")
    sid = lax.axis_index("s")
    wid = sid * NC + cid
    zv = jnp.zeros((16,), jnp.float32)

    def _zero(i, carry):
        zbuf[i, :] = zv
        return carry

    lax.fori_loop(0, 640, _zero, 0)
    pltpu.sync_copy(zbuf, denom_sh.at[pl.ds(sid * 640, 640)])
    pltpu.sync_copy(bacc, bvm)
    plsc.subcore_barrier()

    bv = jnp.maximum(bvm[0, :] + bvm[1, :], 0.0)
    lane8 = lax.iota(jnp.int32, 16) < 8
    rows0 = wid * WROWS

    def _chunk(t, carry):
        r = rows0 + t * 8
        pltpu.sync_copy(srcm.at[pl.ds(r, 8)], sidx)
        pltpu.sync_copy(dstm.at[pl.ds(r, 8)], didx)
        cps = []
        for j in range(8):
            cps.append(pltpu.async_copy(
                atabs.at[sidx.at[j]], g1.at[pl.ds(j * 128, 128)], sem))
            cps.append(pltpu.async_copy(
                atabd.at[didx.at[j]], g2.at[pl.ds(j * 128, 128)], sem))
        for cp in cps:
            cp.wait()

        def _edge(i, c2):
            s = g1[i, :] + g2[i, :]
            a = jnp.where(s >= 0.0, s, 0.2 * s)
            w = jnp.exp(a - bv)
            wbuf[i, :] = jnp.where(lane8, w, 0.0)
            return c2

        lax.fori_loop(0, 1024, _edge, 0)
        for j in range(8):
            pltpu.sync_copy(wbuf.at[pl.ds(j * 128, 128)],
                            denom_sh.at[didx.at[j]], add=True)
        pltpu.sync_copy(wbuf, w_hbm.at[pl.ds(r * 128, 1024)])
        return carry

    lax.fori_loop(0, WROWS // 8, _chunk, 0)
    plsc.subcore_barrier()

    @pl.when(jnp.logical_and(sid == 0, cid == 0))
    def _():
        pltpu.sync_copy(denom_sh, d0_hbm)

    @pl.when(jnp.logical_and(sid == 0, cid == 1))
    def _():
        pltpu.sync_copy(denom_sh, d1_hbm)


@functools.partial(
    pl.kernel,
    out_type=(
        jax.ShapeDtypeStruct((NPAD, NCLS), jnp.float32),
        jax.ShapeDtypeStruct((NPAD, NCLS), jnp.float32),
    ),
    mesh=_sc_mesh,
    compiler_params=pltpu.CompilerParams(use_tc_tiling_on_sc=False),
    scratch_types=[
        pltpu.VMEM((1, 128), jnp.int32),
        pltpu.VMEM((1, 128), jnp.int32),
        pltpu.VMEM((128, DGAT), jnp.float32),
        pltpu.VMEM((128, 16), jnp.float32),
        pltpu.VMEM((128, 16), jnp.float32),
        pltpu.VMEM((128, 16), jnp.float32),
        pltpu.VMEM((128, 16), jnp.float32),
        pltpu.VMEM((128, NCLS), jnp.float32),
        pltpu.VMEM((64, NCLS), jnp.float32),
        pltpu.VMEM_SHARED((NPAD, NCLS), jnp.float32),
        pltpu.SemaphoreType.DMA,
    ],
)
def _sc_b2(srcm, dstm, w_hbm, d0_hbm, d1_hbm, xw_hbm, o0_hbm, o1_hbm,
           sidx, didx, xwbuf, wbuf, g3, g4, cbuf, msg, zbuf, out_sh, sem):
    cid = lax.axis_index("c")
    sid = lax.axis_index("s")
    wid = sid * NC + cid
    zv = jnp.zeros((16,), jnp.float32)

    def _zero(i, carry):
        for q in range(NCLS // 16):
            zbuf[i, pl.ds(q * 16, 16)] = zv
        return carry

    lax.fori_loop(0, 64, _zero, 0)
    for k in range(10):
        pltpu.sync_copy(zbuf, out_sh.at[pl.ds(sid * 640 + k * 64, 64)])
    plsc.subcore_barrier()

    rows0 = wid * WROWS

    def _chunk(t, carry):
        r = rows0 + t
        pltpu.sync_copy(srcm.at[pl.ds(r, 1)], sidx)
        pltpu.sync_copy(dstm.at[pl.ds(r, 1)], didx)
        cp1 = pltpu.async_copy(xw_hbm.at[sidx.at[0]], xwbuf, sem)
        cp2 = pltpu.async_copy(d0_hbm.at[didx.at[0]], g3, sem)
        cp3 = pltpu.async_copy(d1_hbm.at[didx.at[0]], g4, sem)
        cp4 = pltpu.async_copy(w_hbm.at[pl.ds(r * 128, 128)], wbuf, sem)
        cp1.wait()
        cp2.wait()
        cp3.wait()
        cp4.wait()

        def _coef(i, c2):
            dv = g3[i, :] + g4[i, :]
            cbuf[i, :] = wbuf[i, :] / (dv + 1e-16) * (1.0 / HEADS)
            return c2

        lax.fori_loop(0, 128, _coef, 0)

        def _msg(i, c2):
            cv = cbuf[i, :]
            for q in range(NCLS // 16):
                acc = jnp.zeros((16,), jnp.float32)
                for hh in range(HEADS):
                    acc = acc + cv[hh] * xwbuf[i, pl.ds(hh * NCLS + q * 16, 16)]
                msg[i, pl.ds(q * 16, 16)] = acc
            return c2

        lax.fori_loop(0, 128, _msg, 0)
        pltpu.sync_copy(msg, out_sh.at[didx.at[0]], add=True)
        return carry

    lax.fori_loop(0, WROWS, _chunk, 0)
    plsc.subcore_barrier()

    @pl.when(jnp.logical_and(sid == 0, cid == 0))
    def _():
        pltpu.sync_copy(out_sh, o0_hbm)

    @pl.when(jnp.logical_and(sid == 0, cid == 1))
    def _():
        pltpu.sync_copy(out_sh, o1_hbm)


def _comb_body(o0_ref, o1_ref, bg_ref, out_ref):
    out_ref[...] = o0_ref[...] + o1_ref[...] + bg_ref[...]


_comb = pl.pallas_call(
    _comb_body,
    grid=(NBLK,),
    in_specs=[
        pl.BlockSpec((ROWBLK, NCLS), lambda i: (i, 0)),
        pl.BlockSpec((ROWBLK, NCLS), lambda i: (i, 0)),
        pl.BlockSpec((1, NCLS), lambda i: (0, 0)),
    ],
    out_specs=pl.BlockSpec((ROWBLK, NCLS), lambda i: (i, 0)),
    out_shape=jax.ShapeDtypeStruct((NPAD, NCLS), jnp.float32),
)


def kernel(x, edge_index, W_emb1, b_emb1, W_emb2, b_emb2, W_lin, b_lin,
           W_gat, att_src, att_dst, b_gat):
    xp = jnp.pad(x, ((0, NPAD - N_NODES), (0, 0)))
    ei = edge_index.astype(jnp.int32)
    src = jnp.pad(ei[0], (0, EPAD - N_EDGES),
                  constant_values=SENT).reshape(EROWS, 128)
    dst = jnp.pad(ei[1], (0, EPAD - N_EDGES),
                  constant_values=SENT).reshape(EROWS, 128)
    asel = att_src.reshape(1, DGAT)
    dsel = att_dst.reshape(1, DGAT)
    esel = (jnp.arange(DGAT)[:, None] // NCLS
            == jnp.arange(HEADS)[None, :]).astype(jnp.float32)

    xw, atabs, atabd, bacc = _dense(
        xp, W_emb1, b_emb1.reshape(1, -1), W_emb2, b_emb2.reshape(1, -1),
        W_lin, b_lin.reshape(1, -1), W_gat, asel, dsel, esel)
    w, d0, d1 = _sc_b1(src, dst, atabs, atabd, bacc)
    o0, o1 = _sc_b2(src, dst, w, d0, d1, xw)
    out = _comb(o0, o1, b_gat.reshape(1, NCLS))
    return out[:N_NODES]
